# Initial kernel scaffold; baseline (speedup 1.0000x reference)
import jax
import jax.numpy as jnp
from jax.experimental import pallas as pl


def kernel(w, y):
    raise NotImplementedError("write your pallas kernel here")


# speedup vs baseline: 1.7096x; 1.7096x over previous
import functools

import jax
import jax.numpy as jnp
from jax.experimental import pallas as pl
from jax.experimental.pallas import tpu as pltpu

_N, _D = 500000, 128
_BN = 20000


def _loss_kernel(w_ref, y_ref, out_ref):
    i = pl.program_id(0)
    wv = w_ref[...]
    y = y_ref[...]
    inv_nw2 = 1.0 / jnp.sum(wv * wv)
    t = jnp.sum(y * wv, axis=1, keepdims=True)
    c = t * inv_nw2
    res = y - c * wv
    d2 = jnp.sum(res * res, axis=1, keepdims=True)
    x = 1.0 + d2
    a = jnp.log(x + jnp.sqrt(x * x - 1.0))
    part = jnp.sum(a * a, axis=0, keepdims=True)

    @pl.when(i == 0)
    def _():
        out_ref[...] = jnp.zeros_like(out_ref)

    out_ref[...] += part


@jax.jit
def kernel(w, y):
    w2 = w.reshape(1, _D)
    grid = (_N // _BN,)
    out = pl.pallas_call(
        _loss_kernel,
        out_shape=jax.ShapeDtypeStruct((1, 1), jnp.float32),
        grid=grid,
        in_specs=[
            pl.BlockSpec((1, _D), lambda i: (0, 0)),
            pl.BlockSpec((_BN, _D), lambda i: (i, 0)),
        ],
        out_specs=pl.BlockSpec((1, 1), lambda i: (0, 0)),
        compiler_params=pltpu.CompilerParams(
            dimension_semantics=("arbitrary",),
        ),
        name="hyperbolic_lines_loss",
    )(w2, y)
    return out[0, 0]


# speedup vs baseline: 2.4584x; 1.4380x over previous
import functools

import jax
import jax.numpy as jnp
from jax.experimental import pallas as pl
from jax.experimental.pallas import tpu as pltpu

_N, _D = 500000, 128
_BN = 20000


def _loss_kernel(w_ref, y_ref, out_ref):
    i = pl.program_id(0)
    wv = w_ref[...]
    y = y_ref[...]
    inv_nw2 = 1.0 / jnp.sum(wv * wv)
    t = jnp.sum(y * wv, axis=1, keepdims=True)
    c = t * inv_nw2
    res = y - c * wv
    d2 = jnp.sum(res * res, axis=1, keepdims=True)

    K = 32
    g = _BN // K
    d2_b = jnp.broadcast_to(d2, (_BN, K)).reshape(g, K, K)
    r_idx = jax.lax.broadcasted_iota(jnp.int32, (K, K), 0)
    l_idx = jax.lax.broadcasted_iota(jnp.int32, (K, K), 1)
    eye = (r_idx == l_idx).astype(jnp.float32)
    d2_dense = jnp.sum(d2_b * eye[None, :, :], axis=1)

    x = 1.0 + d2_dense
    a = jnp.log(x + jnp.sqrt(x * x - 1.0))
    aa = a * a
    col = jnp.sum(aa, axis=0, keepdims=True)
    part = jnp.sum(col, axis=1, keepdims=True)

    @pl.when(i == 0)
    def _():
        out_ref[...] = jnp.zeros_like(out_ref)

    out_ref[...] += part


@jax.jit
def kernel(w, y):
    w2 = w.reshape(1, _D)
    grid = (_N // _BN,)
    out = pl.pallas_call(
        _loss_kernel,
        out_shape=jax.ShapeDtypeStruct((1, 1), jnp.float32),
        grid=grid,
        in_specs=[
            pl.BlockSpec((1, _D), lambda i: (0, 0)),
            pl.BlockSpec((_BN, _D), lambda i: (i, 0)),
        ],
        out_specs=pl.BlockSpec((1, 1), lambda i: (0, 0)),
        compiler_params=pltpu.CompilerParams(
            dimension_semantics=("arbitrary",),
        ),
        name="hyperbolic_lines_loss",
    )(w2, y)
    return out[0, 0]
